# baseline (speedup 1.0000x reference)
import functools

import jax
import jax.numpy as jnp
from jax import lax
from jax.experimental import pallas as pl
from jax.experimental.pallas import tpu as pltpu
from jax.experimental.pallas import tpu_sc as plsc

ALPHA = 0.1
K = 10
NC, NS = 2, 16
NW = NC * NS
EB = 128
DEPTH = 3


def _mesh():
    return plsc.VectorSubcoreMesh(
        core_axis_name="c", subcore_axis_name="s",
        num_cores=NC, num_subcores=NS)


_SC_PARAMS = pltpu.CompilerParams(needs_layout_passes=False,
                                  use_tc_tiling_on_sc=False)


def _edge_pass(u_sh, acc_sh, sidx_v, didx_v, rows, semg, sems, bps):
    ngrp = bps // DEPTH
    for j in range(DEPTH):
        pltpu.async_copy(u_sh.at[sidx_v.at[j]], rows[j], semg[j])

    def eb(i2, c):
        for bank in range(2):
            bb = bank * DEPTH
            ob = (1 - bank) * DEPTH
            i = i2 * 2 + bank
            b0 = i * DEPTH
            for j in range(DEPTH):
                b = b0 + j
                pltpu.make_async_copy(
                    u_sh.at[sidx_v.at[b]], rows[bb + j],
                    semg[bb + j]).wait()
                pltpu.async_copy(
                    rows[bb + j], acc_sh.at[didx_v.at[b]],
                    sems[bb + j], add=True)
            for j in range(DEPTH):
                pj = ob + j
                pb = (i - 1) * DEPTH + j

                def _wait_prev(pj=pj, pb=pb):
                    pltpu.make_async_copy(
                        rows[pj], acc_sh.at[didx_v.at[pb]],
                        sems[pj]).wait()
                if bank == 0:
                    pl.when(i2 > 0)(_wait_prev)
                else:
                    _wait_prev()
                nb = (i + 1) * DEPTH + j

                @pl.when(nb < bps)
                def _(pj=pj, nb=nb):
                    pltpu.async_copy(u_sh.at[sidx_v.at[nb]],
                                     rows[pj], semg[pj])
        return c
    lax.fori_loop(0, ngrp // 2, eb, 0)
    for j in range(DEPTH):
        b = (ngrp - 1) * DEPTH + j
        pltpu.make_async_copy(
            rows[DEPTH + j], acc_sh.at[didx_v.at[b]],
            sems[DEPTH + j]).wait()


def _scale_rows(dst, src, mult_ch, off, nrows, hw):
    def p(r16, cc):
        m16 = mult_ch[pl.ds(off + r16 * 16, 16)]
        for k in range(16):
            r = r16 * 16 + k
            m = m16[k]
            for g in range(hw // 16):
                sl = pl.ds(g * 16, 16)
                dst[r, sl] = jnp.where(m > 0.0, m * src[r, sl], 0.0)
        return cc
    lax.fori_loop(0, nrows // 16, p, 0)


@functools.lru_cache(maxsize=None)
def _make_sweep(n_pad, bps, hw, n_iter):
    rps = n_pad // NS

    hrp = rps // 2

    def body(u0c_h, seedc_h, dinv_h, sidx_h, didx_h, zc_h,
             sidx_v, didx_v, acc_b, dinv_ch, dva_ch, dvq_ch,
             acc_sh, u_sh, *bufs):
        nb2 = 2 * DEPTH
        rows = bufs[:nb2]
        semg = bufs[nb2:2 * nb2]
        sems = bufs[2 * nb2:3 * nb2]
        cid = lax.axis_index("c")
        sid = lax.axis_index("s")
        r0 = sid * rps
        chunk = pl.ds(r0, rps)
        pltpu.sync_copy(sidx_h.at[sid], sidx_v)
        pltpu.sync_copy(didx_h.at[sid], didx_v)
        pltpu.sync_copy(dinv_h.at[chunk], dinv_ch)
        pltpu.sync_copy(u0c_h.at[cid, chunk], u_sh.at[chunk])
        pltpu.sync_copy(seedc_h.at[cid, chunk], acc_sh.at[chunk])

        def prep(r16, cc):
            sl0 = pl.ds(r16 * 16, 16)
            dv16 = dinv_ch[sl0]
            dva16 = (1.0 - ALPHA) * dv16
            dva_ch[sl0] = dva16
            dvq_ch[sl0] = dv16 * dva16
            return cc
        lax.fori_loop(0, rps // 16, prep, 0)
        plsc.subcore_barrier()

        def it_body(it, c):
            _edge_pass(u_sh, acc_sh, sidx_v, didx_v,
                       rows, semg, sems, bps)
            plsc.subcore_barrier()
            for hf in range(2):
                off = hf * hrp
                hchunk = pl.ds(r0 + off, hrp)
                pltpu.sync_copy(acc_sh.at[hchunk], acc_b)
                pltpu.sync_copy(seedc_h.at[cid, hchunk], acc_sh.at[hchunk])

                @pl.when(it < n_iter - 1)
                def _(off=off, hchunk=hchunk):
                    _scale_rows(acc_b, acc_b, dvq_ch, off, hrp, hw)
                    pltpu.sync_copy(acc_b, u_sh.at[hchunk])

                @pl.when(it == n_iter - 1)
                def _(off=off, hchunk=hchunk):
                    _scale_rows(acc_b, acc_b, dva_ch, off, hrp, hw)
                    pltpu.sync_copy(acc_b, zc_h.at[cid, hchunk])
            plsc.subcore_barrier()
            return c
        lax.fori_loop(0, n_iter, it_body, 0)

    return pl.kernel(
        body,
        out_type=jax.ShapeDtypeStruct((NC, n_pad, hw), jnp.float32),
        mesh=_mesh(),
        compiler_params=_SC_PARAMS,
        scratch_types=[
            pltpu.VMEM((bps, EB), jnp.int32),
            pltpu.VMEM((bps, EB), jnp.int32),
            pltpu.VMEM((rps // 2, hw), jnp.float32),
            pltpu.VMEM((rps,), jnp.float32),
            pltpu.VMEM((rps,), jnp.float32),
            pltpu.VMEM((rps,), jnp.float32),
            pltpu.VMEM_SHARED((n_pad, hw), jnp.float32),
            pltpu.VMEM_SHARED((n_pad, hw), jnp.float32),
        ] + [pltpu.VMEM((EB, hw), jnp.float32)] * (2 * DEPTH)
          + [pltpu.SemaphoreType.DMA] * (4 * DEPTH),
    )


@functools.lru_cache(maxsize=None)
def _make_fprop(n_pad, bps, hw):
    rps = n_pad // NS
    zr = rps // 4

    def body(yc_h, dinv_h, sidx_h, didx_h, out_h,
             sidx_v, didx_v, buf, dinv_ch, zb, acc_sh, u_sh, *bufs):
        nb2 = 2 * DEPTH
        rows = bufs[:nb2]
        semg = bufs[nb2:2 * nb2]
        sems = bufs[2 * nb2:3 * nb2]
        cid = lax.axis_index("c")
        sid = lax.axis_index("s")
        r0 = sid * rps
        chunk = pl.ds(r0, rps)
        pltpu.sync_copy(sidx_h.at[sid], sidx_v)
        pltpu.sync_copy(didx_h.at[sid], didx_v)
        pltpu.sync_copy(yc_h.at[cid, chunk], buf)
        pltpu.sync_copy(dinv_h.at[chunk], dinv_ch)
        _scale_rows(buf, buf, dinv_ch, 0, rps, hw)
        pltpu.sync_copy(buf, u_sh.at[chunk])
        zero = jnp.zeros((16,), jnp.float32)

        def zzb(i, c):
            for g in range(hw // 16):
                zb[i, pl.ds(g * 16, 16)] = zero
            return c
        lax.fori_loop(0, zr, zzb, 0)
        for q in range(4):
            pltpu.sync_copy(zb, acc_sh.at[pl.ds(r0 + q * zr, zr)])
        plsc.subcore_barrier()
        _edge_pass(u_sh, acc_sh, sidx_v, didx_v, rows, semg, sems, bps)
        plsc.subcore_barrier()
        pltpu.sync_copy(acc_sh.at[chunk], buf)
        _scale_rows(buf, buf, dinv_ch, 0, rps, hw)
        pltpu.sync_copy(buf, out_h.at[cid, chunk])

    return pl.kernel(
        body,
        out_type=jax.ShapeDtypeStruct((NC, n_pad, hw), jnp.float32),
        mesh=_mesh(),
        compiler_params=_SC_PARAMS,
        scratch_types=[
            pltpu.VMEM((bps, EB), jnp.int32),
            pltpu.VMEM((bps, EB), jnp.int32),
            pltpu.VMEM((rps, hw), jnp.float32),
            pltpu.VMEM((rps,), jnp.float32),
            pltpu.VMEM((zr, hw), jnp.float32),
            pltpu.VMEM_SHARED((n_pad, hw), jnp.float32),
            pltpu.VMEM_SHARED((n_pad, hw), jnp.float32),
        ] + [pltpu.VMEM((EB, hw), jnp.float32)] * (2 * DEPTH)
          + [pltpu.SemaphoreType.DMA] * (4 * DEPTH),
    )


@functools.lru_cache(maxsize=None)
def _make_deg(n_pad, bps):
    hbps = bps // 2

    def body(didx_hbm, degp_hbm, didx_v, deg_v):
        cid = lax.axis_index("c")
        sid = lax.axis_index("s")
        wid = sid * NC + cid
        pltpu.sync_copy(didx_hbm.at[sid, pl.ds(cid * hbps, hbps)], didx_v)
        zero = jnp.zeros((16,), jnp.float32)

        def zb(i, c):
            deg_v[pl.ds(i * 16, 16)] = zero
            return c
        lax.fori_loop(0, n_pad // 16, zb, 0)
        ones = jnp.ones((16,), jnp.float32)

        def eb(b, c):
            for g in range(EB // 16):
                idx = didx_v[b, pl.ds(g * 16, 16)]
                plsc.addupdate_scatter(deg_v, [idx], ones)
            return c
        lax.fori_loop(0, hbps, eb, 0)
        pltpu.sync_copy(deg_v, degp_hbm.at[wid])

    return pl.kernel(
        body,
        out_type=jax.ShapeDtypeStruct((NW, n_pad), jnp.float32),
        mesh=_mesh(),
        compiler_params=_SC_PARAMS,
        scratch_types=[
            pltpu.VMEM((hbps, EB), jnp.int32),
            pltpu.VMEM((n_pad,), jnp.float32),
        ],
    )


def _mm1_body(x_ref, w_ref, b_ref, hc_ref):
    h = jnp.maximum(
        jnp.dot(x_ref[...], w_ref[...], preferred_element_type=jnp.float32)
        + b_ref[...], 0.0)
    n = h.shape[0]
    half = h.shape[1] // 2
    hc_ref[0, :n, :] = h[:, :half]
    hc_ref[1, :n, :] = h[:, half:]


def _dinv_body(degt_ref, hc_ref, dinv_ref, u0c_ref, seedc_ref, *, n_real):
    deg = jnp.sum(degt_ref[...], axis=1, keepdims=True)
    dinv = lax.rsqrt(jnp.maximum(deg, 1.0))
    row = lax.broadcasted_iota(jnp.int32, dinv.shape, 0)
    dinv = jnp.where(row < n_real, dinv, 0.0)
    dinv_ref[...] = dinv
    hc = hc_ref[...]
    d3 = dinv[None, :, :]
    u0c_ref[...] = jnp.where(d3 > 0.0, d3 * hc, 0.0)
    seedc_ref[...] = jnp.where(
        d3 > 0.0, (ALPHA / (1.0 - ALPHA)) * hc / jnp.maximum(d3, 1e-30),
        0.0)


def _att_body(zc_ref, lt_ref, g_ref, aw1_ref, ab1_ref, aw2_ref, wg_ref,
              yc_ref):
    n = lt_ref.shape[0]
    z = jnp.concatenate([zc_ref[0, :n, :], zc_ref[1, :n, :]], axis=1)
    lt = lt_ref[...]
    aw1 = aw1_ref[...]
    ab1 = ab1_ref[...]
    aw2 = aw2_ref[...]
    wz = jnp.dot(jnp.tanh(
        jnp.dot(z, aw1, preferred_element_type=jnp.float32) + ab1),
        aw2, preferred_element_type=jnp.float32)
    wl = jnp.dot(jnp.tanh(
        jnp.dot(lt, aw1, preferred_element_type=jnp.float32) + ab1),
        aw2, preferred_element_type=jnp.float32)
    m = jnp.maximum(wz, wl)
    ez = jnp.exp(wz - m)
    el = jnp.exp(wl - m)
    emb2 = (ez * z + el * lt) / (ez + el)
    z2 = emb2 * g_ref[...]
    y = jnp.dot(z2, wg_ref[...], preferred_element_type=jnp.float32)
    half = y.shape[1] // 2
    yc_ref[0, :n, :] = y[:, :half]
    yc_ref[1, :n, :] = y[:, half:]


def _final_body(accf_ref, bg_ref, out_ref):
    n = out_ref.shape[0]
    acc = jnp.concatenate([accf_ref[0, :n, :], accf_ref[1, :n, :]], axis=1)
    o = acc + bg_ref[...]
    m = jnp.max(o, axis=1, keepdims=True)
    s = jnp.sum(jnp.exp(o - m), axis=1, keepdims=True)
    out_ref[...] = (o - m) - jnp.log(s)


def kernel(x, edge_index, local_topo, global_topo, W1, b1,
           attW1, attb1, attw2, Wg, bg):
    n, nfeat = x.shape
    e = edge_index.shape[1]
    nhid = W1.shape[1]
    nclass = Wg.shape[1]
    n_pad = -(-(n + 1) // (NS * 16)) * NS * 16
    e_full = e + n
    bps = -(-e_full // (NS * EB))
    bps = -(-bps // (2 * DEPTH)) * 2 * DEPTH
    e_pad = NS * EB * bps

    loop = jnp.arange(n, dtype=jnp.int32)
    fill = jnp.full((e_pad - e_full,), n, jnp.int32)
    sidx = jnp.concatenate([edge_index[0], loop, fill]).reshape(NS, bps, EB)
    didx = jnp.concatenate([edge_index[1], loop, fill]).reshape(NS, bps, EB)

    b1r = b1.reshape(1, nhid)
    ab1r = attb1.reshape(1, -1)
    bgr = bg.reshape(1, nclass)
    g = global_topo.reshape(1, nhid)
    hw_s = nhid // NC
    hw_f = nclass // NC

    f32 = jnp.float32
    hc = pl.pallas_call(
        _mm1_body,
        out_shape=jax.ShapeDtypeStruct((NC, n_pad, hw_s), f32))(x, W1, b1r)

    degp = _make_deg(n_pad, bps)(didx)
    dinv, u0c, seedc = pl.pallas_call(
        functools.partial(_dinv_body, n_real=n),
        out_shape=[jax.ShapeDtypeStruct((n_pad, 1), f32),
                   jax.ShapeDtypeStruct((NC, n_pad, hw_s), f32),
                   jax.ShapeDtypeStruct((NC, n_pad, hw_s), f32)])(
        degp.T, hc)
    dinv_flat = dinv[:, 0]

    zc = _make_sweep(n_pad, bps, hw_s, K)(u0c, seedc, dinv_flat, sidx, didx)

    yc = pl.pallas_call(
        _att_body,
        out_shape=jax.ShapeDtypeStruct((NC, n_pad, hw_f), f32))(
        zc, local_topo, g, attW1, ab1r, attw2, Wg)

    accf = _make_fprop(n_pad, bps, hw_f)(yc, dinv_flat, sidx, didx)
    return pl.pallas_call(
        _final_body,
        out_shape=jax.ShapeDtypeStruct((n, nclass), f32))(accf, bgr)


# speedup vs baseline: 1.0049x; 1.0049x over previous
import functools

import jax
import jax.numpy as jnp
from jax import lax
from jax.experimental import pallas as pl
from jax.experimental.pallas import tpu as pltpu
from jax.experimental.pallas import tpu_sc as plsc

ALPHA = 0.1
K = 10
NC, NS = 2, 16
NW = NC * NS
EB = 128
DEPTH = 4


def _mesh():
    return plsc.VectorSubcoreMesh(
        core_axis_name="c", subcore_axis_name="s",
        num_cores=NC, num_subcores=NS)


_SC_PARAMS = pltpu.CompilerParams(needs_layout_passes=False,
                                  use_tc_tiling_on_sc=False)


def _edge_pass(u_sh, acc_sh, sidx_v, didx_v, rows, semg, sems, bps):
    ngrp = bps // DEPTH
    for j in range(DEPTH):
        pltpu.async_copy(u_sh.at[sidx_v.at[j]], rows[j], semg[j])

    def eb(i2, c):
        for bank in range(2):
            bb = bank * DEPTH
            ob = (1 - bank) * DEPTH
            i = i2 * 2 + bank
            b0 = i * DEPTH
            for j in range(DEPTH):
                b = b0 + j
                pltpu.make_async_copy(
                    u_sh.at[sidx_v.at[b]], rows[bb + j],
                    semg[bb + j]).wait()
                pltpu.async_copy(
                    rows[bb + j], acc_sh.at[didx_v.at[b]],
                    sems[bb + j], add=True)
            for j in range(DEPTH):
                pj = ob + j
                pb = (i - 1) * DEPTH + j

                def _wait_prev(pj=pj, pb=pb):
                    pltpu.make_async_copy(
                        rows[pj], acc_sh.at[didx_v.at[pb]],
                        sems[pj]).wait()
                if bank == 0:
                    pl.when(i2 > 0)(_wait_prev)
                else:
                    _wait_prev()
                nb = (i + 1) * DEPTH + j

                @pl.when(nb < bps)
                def _(pj=pj, nb=nb):
                    pltpu.async_copy(u_sh.at[sidx_v.at[nb]],
                                     rows[pj], semg[pj])
        return c
    lax.fori_loop(0, ngrp // 2, eb, 0)
    for j in range(DEPTH):
        b = (ngrp - 1) * DEPTH + j
        pltpu.make_async_copy(
            rows[DEPTH + j], acc_sh.at[didx_v.at[b]],
            sems[DEPTH + j]).wait()


def _scale_rows(dst, src, mult_ch, off, nrows, hw):
    def p(r16, cc):
        m16 = mult_ch[pl.ds(off + r16 * 16, 16)]
        for k in range(16):
            r = r16 * 16 + k
            m = m16[k]
            for g in range(hw // 16):
                sl = pl.ds(g * 16, 16)
                dst[r, sl] = jnp.where(m > 0.0, m * src[r, sl], 0.0)
        return cc
    lax.fori_loop(0, nrows // 16, p, 0)


@functools.lru_cache(maxsize=None)
def _make_sweep(n_pad, bps, hw, n_iter):
    rps = n_pad // NS

    hrp = rps // 2

    def body(u0c_h, seedc_h, dinv_h, sidx_h, didx_h, zc_h,
             sidx_v, didx_v, acc_b, dinv_ch, dva_ch, dvq_ch,
             acc_sh, u_sh, *bufs):
        nb2 = 2 * DEPTH
        rows = bufs[:nb2]
        semg = bufs[nb2:2 * nb2]
        sems = bufs[2 * nb2:3 * nb2]
        cid = lax.axis_index("c")
        sid = lax.axis_index("s")
        r0 = sid * rps
        chunk = pl.ds(r0, rps)
        pltpu.sync_copy(sidx_h.at[sid], sidx_v)
        pltpu.sync_copy(didx_h.at[sid], didx_v)
        pltpu.sync_copy(dinv_h.at[chunk], dinv_ch)
        pltpu.sync_copy(u0c_h.at[cid, chunk], u_sh.at[chunk])
        pltpu.sync_copy(seedc_h.at[cid, chunk], acc_sh.at[chunk])

        def prep(r16, cc):
            sl0 = pl.ds(r16 * 16, 16)
            dv16 = dinv_ch[sl0]
            dva16 = (1.0 - ALPHA) * dv16
            dva_ch[sl0] = dva16
            dvq_ch[sl0] = dv16 * dva16
            return cc
        lax.fori_loop(0, rps // 16, prep, 0)
        plsc.subcore_barrier()

        def it_body(it, c):
            _edge_pass(u_sh, acc_sh, sidx_v, didx_v,
                       rows, semg, sems, bps)
            plsc.subcore_barrier()
            for hf in range(2):
                off = hf * hrp
                hchunk = pl.ds(r0 + off, hrp)
                pltpu.sync_copy(acc_sh.at[hchunk], acc_b)
                pltpu.sync_copy(seedc_h.at[cid, hchunk], acc_sh.at[hchunk])

                @pl.when(it < n_iter - 1)
                def _(off=off, hchunk=hchunk):
                    _scale_rows(acc_b, acc_b, dvq_ch, off, hrp, hw)
                    pltpu.sync_copy(acc_b, u_sh.at[hchunk])

                @pl.when(it == n_iter - 1)
                def _(off=off, hchunk=hchunk):
                    _scale_rows(acc_b, acc_b, dva_ch, off, hrp, hw)
                    pltpu.sync_copy(acc_b, zc_h.at[cid, hchunk])
            plsc.subcore_barrier()
            return c
        lax.fori_loop(0, n_iter, it_body, 0)

    return pl.kernel(
        body,
        out_type=jax.ShapeDtypeStruct((NC, n_pad, hw), jnp.float32),
        mesh=_mesh(),
        compiler_params=_SC_PARAMS,
        scratch_types=[
            pltpu.VMEM((bps, EB), jnp.int32),
            pltpu.VMEM((bps, EB), jnp.int32),
            pltpu.VMEM((rps // 2, hw), jnp.float32),
            pltpu.VMEM((rps,), jnp.float32),
            pltpu.VMEM((rps,), jnp.float32),
            pltpu.VMEM((rps,), jnp.float32),
            pltpu.VMEM_SHARED((n_pad, hw), jnp.float32),
            pltpu.VMEM_SHARED((n_pad, hw), jnp.float32),
        ] + [pltpu.VMEM((EB, hw), jnp.float32)] * (2 * DEPTH)
          + [pltpu.SemaphoreType.DMA] * (4 * DEPTH),
    )


@functools.lru_cache(maxsize=None)
def _make_fprop(n_pad, bps, hw):
    rps = n_pad // NS
    zr = rps // 4

    def body(yc_h, dinv_h, sidx_h, didx_h, out_h,
             sidx_v, didx_v, buf, dinv_ch, zb, acc_sh, u_sh, *bufs):
        nb2 = 2 * DEPTH
        rows = bufs[:nb2]
        semg = bufs[nb2:2 * nb2]
        sems = bufs[2 * nb2:3 * nb2]
        cid = lax.axis_index("c")
        sid = lax.axis_index("s")
        r0 = sid * rps
        chunk = pl.ds(r0, rps)
        pltpu.sync_copy(sidx_h.at[sid], sidx_v)
        pltpu.sync_copy(didx_h.at[sid], didx_v)
        pltpu.sync_copy(yc_h.at[cid, chunk], buf)
        pltpu.sync_copy(dinv_h.at[chunk], dinv_ch)
        _scale_rows(buf, buf, dinv_ch, 0, rps, hw)
        pltpu.sync_copy(buf, u_sh.at[chunk])
        zero = jnp.zeros((16,), jnp.float32)

        def zzb(i, c):
            for g in range(hw // 16):
                zb[i, pl.ds(g * 16, 16)] = zero
            return c
        lax.fori_loop(0, zr, zzb, 0)
        for q in range(4):
            pltpu.sync_copy(zb, acc_sh.at[pl.ds(r0 + q * zr, zr)])
        plsc.subcore_barrier()
        _edge_pass(u_sh, acc_sh, sidx_v, didx_v, rows, semg, sems, bps)
        plsc.subcore_barrier()
        pltpu.sync_copy(acc_sh.at[chunk], buf)
        _scale_rows(buf, buf, dinv_ch, 0, rps, hw)
        pltpu.sync_copy(buf, out_h.at[cid, chunk])

    return pl.kernel(
        body,
        out_type=jax.ShapeDtypeStruct((NC, n_pad, hw), jnp.float32),
        mesh=_mesh(),
        compiler_params=_SC_PARAMS,
        scratch_types=[
            pltpu.VMEM((bps, EB), jnp.int32),
            pltpu.VMEM((bps, EB), jnp.int32),
            pltpu.VMEM((rps, hw), jnp.float32),
            pltpu.VMEM((rps,), jnp.float32),
            pltpu.VMEM((zr, hw), jnp.float32),
            pltpu.VMEM_SHARED((n_pad, hw), jnp.float32),
            pltpu.VMEM_SHARED((n_pad, hw), jnp.float32),
        ] + [pltpu.VMEM((EB, hw), jnp.float32)] * (2 * DEPTH)
          + [pltpu.SemaphoreType.DMA] * (4 * DEPTH),
    )


@functools.lru_cache(maxsize=None)
def _make_deg(n_pad, bps):
    hbps = bps // 2

    def body(didx_hbm, degp_hbm, didx_v, deg_v):
        cid = lax.axis_index("c")
        sid = lax.axis_index("s")
        wid = sid * NC + cid
        pltpu.sync_copy(didx_hbm.at[sid, pl.ds(cid * hbps, hbps)], didx_v)
        zero = jnp.zeros((16,), jnp.float32)

        def zb(i, c):
            deg_v[pl.ds(i * 16, 16)] = zero
            return c
        lax.fori_loop(0, n_pad // 16, zb, 0)
        ones = jnp.ones((16,), jnp.float32)

        def eb(b, c):
            for g in range(EB // 16):
                idx = didx_v[b, pl.ds(g * 16, 16)]
                plsc.addupdate_scatter(deg_v, [idx], ones)
            return c
        lax.fori_loop(0, hbps, eb, 0)
        pltpu.sync_copy(deg_v, degp_hbm.at[wid])

    return pl.kernel(
        body,
        out_type=jax.ShapeDtypeStruct((NW, n_pad), jnp.float32),
        mesh=_mesh(),
        compiler_params=_SC_PARAMS,
        scratch_types=[
            pltpu.VMEM((hbps, EB), jnp.int32),
            pltpu.VMEM((n_pad,), jnp.float32),
        ],
    )


def _mm1_body(x_ref, w_ref, b_ref, hc_ref):
    h = jnp.maximum(
        jnp.dot(x_ref[...], w_ref[...], preferred_element_type=jnp.float32)
        + b_ref[...], 0.0)
    n = h.shape[0]
    half = h.shape[1] // 2
    hc_ref[0, :n, :] = h[:, :half]
    hc_ref[1, :n, :] = h[:, half:]


def _dinv_body(degt_ref, hc_ref, dinv_ref, u0c_ref, seedc_ref, *, n_real):
    deg = jnp.sum(degt_ref[...], axis=1, keepdims=True)
    dinv = lax.rsqrt(jnp.maximum(deg, 1.0))
    row = lax.broadcasted_iota(jnp.int32, dinv.shape, 0)
    dinv = jnp.where(row < n_real, dinv, 0.0)
    dinv_ref[...] = dinv
    hc = hc_ref[...]
    d3 = dinv[None, :, :]
    u0c_ref[...] = jnp.where(d3 > 0.0, d3 * hc, 0.0)
    seedc_ref[...] = jnp.where(
        d3 > 0.0, (ALPHA / (1.0 - ALPHA)) * hc / jnp.maximum(d3, 1e-30),
        0.0)


def _att_body(zc_ref, lt_ref, g_ref, aw1_ref, ab1_ref, aw2_ref, wg_ref,
              yc_ref):
    n = lt_ref.shape[0]
    z = jnp.concatenate([zc_ref[0, :n, :], zc_ref[1, :n, :]], axis=1)
    lt = lt_ref[...]
    aw1 = aw1_ref[...]
    ab1 = ab1_ref[...]
    aw2 = aw2_ref[...]
    wz = jnp.dot(jnp.tanh(
        jnp.dot(z, aw1, preferred_element_type=jnp.float32) + ab1),
        aw2, preferred_element_type=jnp.float32)
    wl = jnp.dot(jnp.tanh(
        jnp.dot(lt, aw1, preferred_element_type=jnp.float32) + ab1),
        aw2, preferred_element_type=jnp.float32)
    m = jnp.maximum(wz, wl)
    ez = jnp.exp(wz - m)
    el = jnp.exp(wl - m)
    emb2 = (ez * z + el * lt) / (ez + el)
    z2 = emb2 * g_ref[...]
    y = jnp.dot(z2, wg_ref[...], preferred_element_type=jnp.float32)
    half = y.shape[1] // 2
    yc_ref[0, :n, :] = y[:, :half]
    yc_ref[1, :n, :] = y[:, half:]


def _final_body(accf_ref, bg_ref, out_ref):
    n = out_ref.shape[0]
    acc = jnp.concatenate([accf_ref[0, :n, :], accf_ref[1, :n, :]], axis=1)
    o = acc + bg_ref[...]
    m = jnp.max(o, axis=1, keepdims=True)
    s = jnp.sum(jnp.exp(o - m), axis=1, keepdims=True)
    out_ref[...] = (o - m) - jnp.log(s)


def kernel(x, edge_index, local_topo, global_topo, W1, b1,
           attW1, attb1, attw2, Wg, bg):
    n, nfeat = x.shape
    e = edge_index.shape[1]
    nhid = W1.shape[1]
    nclass = Wg.shape[1]
    n_pad = -(-(n + 1) // (NS * 16)) * NS * 16
    e_full = e + n
    bps = -(-e_full // (NS * EB))
    bps = -(-bps // (2 * DEPTH)) * 2 * DEPTH
    e_pad = NS * EB * bps

    loop = jnp.arange(n, dtype=jnp.int32)
    fill = jnp.full((e_pad - e_full,), n, jnp.int32)
    sidx = jnp.concatenate([edge_index[0], loop, fill]).reshape(NS, bps, EB)
    didx = jnp.concatenate([edge_index[1], loop, fill]).reshape(NS, bps, EB)

    b1r = b1.reshape(1, nhid)
    ab1r = attb1.reshape(1, -1)
    bgr = bg.reshape(1, nclass)
    g = global_topo.reshape(1, nhid)
    hw_s = nhid // NC
    hw_f = nclass // NC

    f32 = jnp.float32
    hc = pl.pallas_call(
        _mm1_body,
        out_shape=jax.ShapeDtypeStruct((NC, n_pad, hw_s), f32))(x, W1, b1r)

    degp = _make_deg(n_pad, bps)(didx)
    dinv, u0c, seedc = pl.pallas_call(
        functools.partial(_dinv_body, n_real=n),
        out_shape=[jax.ShapeDtypeStruct((n_pad, 1), f32),
                   jax.ShapeDtypeStruct((NC, n_pad, hw_s), f32),
                   jax.ShapeDtypeStruct((NC, n_pad, hw_s), f32)])(
        degp.T, hc)
    dinv_flat = dinv[:, 0]

    zc = _make_sweep(n_pad, bps, hw_s, K)(u0c, seedc, dinv_flat, sidx, didx)

    yc = pl.pallas_call(
        _att_body,
        out_shape=jax.ShapeDtypeStruct((NC, n_pad, hw_f), f32))(
        zc, local_topo, g, attW1, ab1r, attw2, Wg)

    accf = _make_fprop(n_pad, bps, hw_f)(yc, dinv_flat, sidx, didx)
    return pl.pallas_call(
        _final_body,
        out_shape=jax.ShapeDtypeStruct((n, nclass), f32))(accf, bgr)
